# baseline (speedup 1.0000x reference)
import functools

import jax
import jax.numpy as jnp
import numpy as np
from jax import lax
from jax.experimental import pallas as pl
from jax.experimental.pallas import tpu as pltpu
from jax.experimental.pallas import tpu_sc as plsc

N = 10000
E = 320000
D_IN = 128
HID = 16

NC = 2
NS = 16
NW = NC * NS
LANES = 16

EB = 128
NBLK = 80
EPW = NBLK * EB
E_PAD = EPW * NW
NROWS = 10240
ROWS_PER_TILE = NROWS // NS
ACC_W = 2 * HID

_GDN = lax.GatherDimensionNumbers(
    offset_dims=(), collapsed_slice_dims=(0,), start_index_map=(0,))


def _shuf(v, perm):
  return lax.gather(v, perm.reshape(LANES, 1), dimension_numbers=_GDN,
                    slice_sizes=(1,),
                    mode=lax.GatherScatterMode.PROMISE_IN_BOUNDS)


def _edge_kernel_body(xors, srcp, dstp, xl, xr, atth, acc_out,
                      attb, sidx, didx, xlb0, xrb0, xlb1, xrb1,
                      em0, em1, zbuf, acc_sh, semg0, semg1, sems0, sems1):
  c = lax.axis_index("c")
  s = lax.axis_index("s")
  wid = s * NC + c

  xlbs = (xlb0, xlb1)
  xrbs = (xrb0, xrb1)
  ems = (em0, em1)
  semgs = (semg0, semg1)
  semss = (sems0, sems1)

  zero = jnp.zeros((LANES,), jnp.float32)

  def zb(i, carry):
    zbuf[i, 0:LANES] = zero
    zbuf[i, LANES:ACC_W] = zero
    return carry

  lax.fori_loop(0, ROWS_PER_TILE // 2, zb, 0)
  zbase = s * ROWS_PER_TILE
  half = ROWS_PER_TILE // 2
  pltpu.sync_copy(zbuf, acc_sh.at[pl.ds(zbase, half)])
  pltpu.sync_copy(zbuf, acc_sh.at[pl.ds(zbase + half, half)])
  pltpu.sync_copy(atth, attb)
  pltpu.sync_copy(srcp.at[wid], sidx)
  pltpu.sync_copy(dstp.at[wid], didx)
  plsc.subcore_barrier()

  att = attb[...]
  att5 = att * jnp.float32(0.2)
  perms = [jnp.arange(LANES, dtype=jnp.int32) ^ x for x in xors]

  def gather_start(g, slot):
    pltpu.make_async_copy(xl.at[sidx.at[g]], xlbs[slot], semgs[slot]).start()
    pltpu.make_async_copy(xr.at[didx.at[g]], xrbs[slot], semgs[slot]).start()

  def gather_wait(slot):
    pltpu.make_async_copy(xl.at[sidx.at[0]], xlbs[slot], semgs[slot]).wait()
    pltpu.make_async_copy(xr.at[didx.at[0]], xrbs[slot], semgs[slot]).wait()

  def scatter_start(g, slot):
    pltpu.make_async_copy(ems[slot], acc_sh.at[didx.at[g]],
                          semss[slot]).start(add=True)

  def scatter_wait(slot):
    pltpu.make_async_copy(ems[slot], acc_sh.at[didx.at[0]],
                          semss[slot]).wait()

  def compute(slot):
    xlb, xrb, em = xlbs[slot], xrbs[slot], ems[slot]

    def edge(i, ecarry):
      vl = xlb[i, :]
      sv = vl + xrb[i, :]
      p = jnp.where(sv > 0, sv * att, sv * att5)
      for perm in perms:
        p = p + _shuf(p, perm)
      ex = jnp.exp(p)
      em[i, 0:LANES] = ex
      em[i, LANES:ACC_W] = ex * vl
      return ecarry

    lax.fori_loop(0, EB, edge, 0)

  gather_start(0, 0)

  def macro(m, carry):
    g0 = 2 * m
    gather_start(g0 + 1, 1)
    gather_wait(0)

    @pl.when(m > 0)
    def _():
      scatter_wait(0)

    compute(0)
    scatter_start(g0, 0)

    @pl.when(g0 + 2 < NBLK)
    def _():
      gather_start(g0 + 2, 0)

    gather_wait(1)

    @pl.when(m > 0)
    def _():
      scatter_wait(1)

    compute(1)
    scatter_start(g0 + 1, 1)
    return carry

  lax.fori_loop(0, NBLK // 2, macro, 0)
  scatter_wait(0)
  scatter_wait(1)
  plsc.subcore_barrier()

  pltpu.sync_copy(acc_sh.at[pl.ds(zbase, ROWS_PER_TILE)],
                  acc_out.at[c, pl.ds(zbase, ROWS_PER_TILE)])


def _make_edge_kernel(xors):
  mesh = plsc.VectorSubcoreMesh(core_axis_name="c", subcore_axis_name="s",
                                num_cores=NC, num_subcores=NS)
  return pl.kernel(
      functools.partial(_edge_kernel_body, xors),
      out_type=jax.ShapeDtypeStruct((NC, NROWS, ACC_W), jnp.float32),
      mesh=mesh,
      scratch_types=[
          pltpu.VMEM((LANES,), jnp.float32),
          pltpu.VMEM((NBLK, EB), jnp.int32),
          pltpu.VMEM((NBLK, EB), jnp.int32),
          pltpu.VMEM((EB, HID), jnp.float32),
          pltpu.VMEM((EB, HID), jnp.float32),
          pltpu.VMEM((EB, HID), jnp.float32),
          pltpu.VMEM((EB, HID), jnp.float32),
          pltpu.VMEM((EB, ACC_W), jnp.float32),
          pltpu.VMEM((EB, ACC_W), jnp.float32),
          pltpu.VMEM((ROWS_PER_TILE // 2, ACC_W), jnp.float32),
          pltpu.VMEM_SHARED((NROWS, ACC_W), jnp.float32),
          pltpu.SemaphoreType.DMA,
          pltpu.SemaphoreType.DMA,
          pltpu.SemaphoreType.DMA,
          pltpu.SemaphoreType.DMA,
      ],
      compiler_params=pltpu.CompilerParams(use_tc_tiling_on_sc=False),
      name="gat_edge_pass",
  )


_edge_l1 = _make_edge_kernel((1, 2))
_edge_l2 = _make_edge_kernel((1, 2, 4, 8))

_G1 = np.kron(np.eye(4, dtype=np.float32), np.ones((4, 4), np.float32))
_G2 = np.ones((HID, HID), np.float32)


def _leaky(v):
  return jnp.where(v > 0, v, v * jnp.float32(0.2))


def _proj1_body(x_ref, w_ref, b_ref, ol_ref, or_ref):
  acc = jnp.dot(x_ref[...], w_ref[...],
                preferred_element_type=jnp.float32) + b_ref[...]
  ol_ref[...] = acc[:, :HID]
  or_ref[...] = acc[:, HID:]


def _proj1(xpad, wcat, bcat):
  return pl.pallas_call(
      _proj1_body,
      out_shape=[
          jax.ShapeDtypeStruct((NROWS, HID), jnp.float32),
          jax.ShapeDtypeStruct((NROWS, HID), jnp.float32),
      ],
  )(xpad, wcat, bcat)


def _fuse_body(acc_ref, xl_ref, xr_ref, att_ref, g_ref, b1_ref, w_ref,
               b2_ref, ol_ref, or_ref):
  xl = xl_ref[...]
  lg = jnp.dot(_leaky(xl + xr_ref[...]) * att_ref[...], g_ref[...],
               preferred_element_type=jnp.float32)
  ex = jnp.exp(lg)
  den = acc_ref[0, :, :HID] + acc_ref[1, :, :HID] + ex
  msg = acc_ref[0, :, HID:] + acc_ref[1, :, HID:] + ex * xl
  h = jnp.maximum(msg / (den + 1e-16) + b1_ref[...], 0.0)
  acc = jnp.dot(h, w_ref[...], preferred_element_type=jnp.float32) + b2_ref[...]
  ol_ref[...] = acc[:, :HID]
  or_ref[...] = acc[:, HID:]


def _fuse(acc1, xl1, xr1, att1v, bias1, wcat2, bcat2):
  return pl.pallas_call(
      _fuse_body,
      out_shape=[
          jax.ShapeDtypeStruct((NROWS, HID), jnp.float32),
          jax.ShapeDtypeStruct((NROWS, HID), jnp.float32),
      ],
  )(acc1, xl1, xr1, att1v, _G1, bias1, wcat2, bcat2)


def _final_body(acc_ref, xl_ref, xr_ref, att_ref, g_ref, b_ref, o_ref):
  xl = xl_ref[...]
  lg = jnp.dot(_leaky(xl + xr_ref[...]) * att_ref[...], g_ref[...],
               preferred_element_type=jnp.float32)
  ex = jnp.exp(lg)
  den = acc_ref[0, :, :HID] + acc_ref[1, :, :HID] + ex
  msg = acc_ref[0, :, HID:] + acc_ref[1, :, HID:] + ex * xl
  out = msg / (den + 1e-16) + b_ref[...]
  o_ref[...] = out[:N]


def _final(acc2, xl2, xr2, att2v, bias2):
  return pl.pallas_call(
      _final_body,
      out_shape=jax.ShapeDtypeStruct((N, HID), jnp.float32),
  )(acc2, xl2, xr2, att2v, _G2, bias2)


@jax.jit
def _impl(x, edge_index, Wl1, bl1, Wr1, br1, att1, bias1,
          Wl2, bl2, Wr2, br2, att2, bias2):
  srcp = jnp.pad(edge_index[0], (0, E_PAD - E),
                 constant_values=N).reshape(NW, NBLK, EB)
  dstp = jnp.pad(edge_index[1], (0, E_PAD - E),
                 constant_values=N).reshape(NW, NBLK, EB)

  xpad = jnp.pad(x, ((0, NROWS - N), (0, 0)))
  w1 = jnp.concatenate([Wl1, Wr1], axis=1)
  b1 = jnp.concatenate([bl1, br1]).reshape(1, 2 * HID)
  xl1, xr1 = _proj1(xpad, w1, b1)

  att1v = att1.reshape(1, HID)
  acc1 = _edge_l1(srcp, dstp, xl1, xr1, att1.reshape(HID))

  w2 = jnp.concatenate([Wl2, Wr2], axis=1)
  b2 = jnp.concatenate([bl2, br2]).reshape(1, 2 * HID)
  xl2, xr2 = _fuse(acc1, xl1, xr1, att1v, bias1.reshape(1, HID), w2, b2)

  att2v = att2.reshape(1, HID)
  acc2 = _edge_l2(srcp, dstp, xl2, xr2, att2.reshape(HID))

  return _final(acc2, xl2, xr2, att2v, bias2.reshape(1, HID))


def kernel(x, edge_index, Wl1, bl1, Wr1, br1, att1, bias1,
           Wl2, bl2, Wr2, br2, att2, bias2):
  return _impl(x, edge_index, Wl1, bl1, Wr1, br1, att1, bias1,
               Wl2, bl2, Wr2, br2, att2, bias2)


# speedup vs baseline: 1.9955x; 1.9955x over previous
import functools

import jax
import jax.numpy as jnp
import numpy as np
from jax import lax
from jax.experimental import pallas as pl
from jax.experimental.pallas import tpu as pltpu
from jax.experimental.pallas import tpu_sc as plsc

N = 10000
E = 320000
D_IN = 128
HID = 16

NC = 2
NS = 16
NW = NC * NS
LANES = 16

EB = 128
NBLK = 80
EPW = NBLK * EB
E_PAD = EPW * NW
NROWS = 10240
ROWS_PER_TILE = NROWS // NS
ACC_W = 2 * HID

_GDN = lax.GatherDimensionNumbers(
    offset_dims=(), collapsed_slice_dims=(0,), start_index_map=(0,))


def _shuf(v, perm):
  return lax.gather(v, perm.reshape(LANES, 1), dimension_numbers=_GDN,
                    slice_sizes=(1,),
                    mode=lax.GatherScatterMode.PROMISE_IN_BOUNDS)


def _edge_kernel_body(xors, srcp, dstp, xl, xr, atth, acc_out,
                      attb, sidx, didx, xlb0, xrb0, xlb1, xrb1,
                      exb0, msgb0, exb1, msgb1, zbuf, den_sh, msg_sh,
                      semg0, semg1, sems0, sems1):
  c = lax.axis_index("c")
  s = lax.axis_index("s")
  wid = s * NC + c

  xlbs = (xlb0, xlb1)
  xrbs = (xrb0, xrb1)
  exbs = (exb0, exb1)
  msgbs = (msgb0, msgb1)
  semgs = (semg0, semg1)
  semss = (sems0, sems1)

  zero = jnp.zeros((LANES,), jnp.float32)

  def zb(i, carry):
    zbuf[i, :] = zero
    return carry

  lax.fori_loop(0, ROWS_PER_TILE // 2, zb, 0)
  zbase = s * ROWS_PER_TILE
  half = ROWS_PER_TILE // 2
  pltpu.sync_copy(zbuf, den_sh.at[pl.ds(zbase, half)])
  pltpu.sync_copy(zbuf, den_sh.at[pl.ds(zbase + half, half)])
  pltpu.sync_copy(zbuf, msg_sh.at[pl.ds(zbase, half)])
  pltpu.sync_copy(zbuf, msg_sh.at[pl.ds(zbase + half, half)])
  pltpu.sync_copy(atth, attb)
  pltpu.sync_copy(srcp.at[wid], sidx)
  pltpu.sync_copy(dstp.at[wid], didx)
  plsc.subcore_barrier()

  att = attb[...]
  att5 = att * jnp.float32(0.2)
  perms = [jnp.arange(LANES, dtype=jnp.int32) ^ x for x in xors]

  def gather_start(g, slot):
    pltpu.make_async_copy(xl.at[sidx.at[g]], xlbs[slot], semgs[slot]).start()
    pltpu.make_async_copy(xr.at[didx.at[g]], xrbs[slot], semgs[slot]).start()

  def gather_wait(slot):
    pltpu.make_async_copy(xl.at[sidx.at[0]], xlbs[slot], semgs[slot]).wait()
    pltpu.make_async_copy(xr.at[didx.at[0]], xrbs[slot], semgs[slot]).wait()

  def scatter_start(g, slot):
    pltpu.make_async_copy(exbs[slot], den_sh.at[didx.at[g]],
                          semss[slot]).start(add=True)
    pltpu.make_async_copy(msgbs[slot], msg_sh.at[didx.at[g]],
                          semss[slot]).start(add=True)

  def scatter_wait(slot):
    pltpu.make_async_copy(exbs[slot], den_sh.at[didx.at[0]],
                          semss[slot]).wait()
    pltpu.make_async_copy(msgbs[slot], msg_sh.at[didx.at[0]],
                          semss[slot]).wait()

  def compute(slot):
    xlb, xrb, exb, msgb = xlbs[slot], xrbs[slot], exbs[slot], msgbs[slot]

    def edge(i, ecarry):
      vl = xlb[i, :]
      sv = vl + xrb[i, :]
      p = jnp.where(sv > 0, sv * att, sv * att5)
      for perm in perms:
        p = p + _shuf(p, perm)
      ex = jnp.exp(p)
      exb[i, :] = ex
      msgb[i, :] = ex * vl
      return ecarry

    lax.fori_loop(0, EB, edge, 0)

  gather_start(0, 0)

  def macro(m, carry):
    g0 = 2 * m
    gather_start(g0 + 1, 1)
    gather_wait(0)

    @pl.when(m > 0)
    def _():
      scatter_wait(0)

    compute(0)
    scatter_start(g0, 0)

    @pl.when(g0 + 2 < NBLK)
    def _():
      gather_start(g0 + 2, 0)

    gather_wait(1)

    @pl.when(m > 0)
    def _():
      scatter_wait(1)

    compute(1)
    scatter_start(g0 + 1, 1)
    return carry

  lax.fori_loop(0, NBLK // 2, macro, 0)
  scatter_wait(0)
  scatter_wait(1)
  plsc.subcore_barrier()

  pltpu.sync_copy(den_sh.at[pl.ds(zbase, ROWS_PER_TILE)],
                  acc_out.at[c, 0, pl.ds(zbase, ROWS_PER_TILE)])
  pltpu.sync_copy(msg_sh.at[pl.ds(zbase, ROWS_PER_TILE)],
                  acc_out.at[c, 1, pl.ds(zbase, ROWS_PER_TILE)])


def _make_edge_kernel(xors):
  mesh = plsc.VectorSubcoreMesh(core_axis_name="c", subcore_axis_name="s",
                                num_cores=NC, num_subcores=NS)
  return pl.kernel(
      functools.partial(_edge_kernel_body, xors),
      out_type=jax.ShapeDtypeStruct((NC, 2, NROWS, HID), jnp.float32),
      mesh=mesh,
      scratch_types=[
          pltpu.VMEM((LANES,), jnp.float32),
          pltpu.VMEM((NBLK, EB), jnp.int32),
          pltpu.VMEM((NBLK, EB), jnp.int32),
          pltpu.VMEM((EB, HID), jnp.float32),
          pltpu.VMEM((EB, HID), jnp.float32),
          pltpu.VMEM((EB, HID), jnp.float32),
          pltpu.VMEM((EB, HID), jnp.float32),
          pltpu.VMEM((EB, HID), jnp.float32),
          pltpu.VMEM((EB, HID), jnp.float32),
          pltpu.VMEM((EB, HID), jnp.float32),
          pltpu.VMEM((EB, HID), jnp.float32),
          pltpu.VMEM((ROWS_PER_TILE // 2, HID), jnp.float32),
          pltpu.VMEM_SHARED((NROWS, HID), jnp.float32),
          pltpu.VMEM_SHARED((NROWS, HID), jnp.float32),
          pltpu.SemaphoreType.DMA,
          pltpu.SemaphoreType.DMA,
          pltpu.SemaphoreType.DMA,
          pltpu.SemaphoreType.DMA,
      ],
      compiler_params=pltpu.CompilerParams(use_tc_tiling_on_sc=False),
      name="gat_edge_pass",
  )


_edge_l1 = _make_edge_kernel((1, 2))
_edge_l2 = _make_edge_kernel((1, 2, 4, 8))

_G1 = np.kron(np.eye(4, dtype=np.float32), np.ones((4, 4), np.float32))
_G2 = np.ones((HID, HID), np.float32)


def _leaky(v):
  return jnp.where(v > 0, v, v * jnp.float32(0.2))


def _proj1_body(x_ref, w_ref, b_ref, ol_ref, or_ref):
  acc = jnp.dot(x_ref[...], w_ref[...],
                preferred_element_type=jnp.float32) + b_ref[...]
  ol_ref[...] = acc[:, :HID]
  or_ref[...] = acc[:, HID:]


def _proj1(xpad, wcat, bcat):
  return pl.pallas_call(
      _proj1_body,
      out_shape=[
          jax.ShapeDtypeStruct((NROWS, HID), jnp.float32),
          jax.ShapeDtypeStruct((NROWS, HID), jnp.float32),
      ],
  )(xpad, wcat, bcat)


def _fuse_body(acc_ref, xl_ref, xr_ref, att_ref, g_ref, b1_ref, w_ref,
               b2_ref, ol_ref, or_ref):
  xl = xl_ref[...]
  lg = jnp.dot(_leaky(xl + xr_ref[...]) * att_ref[...], g_ref[...],
               preferred_element_type=jnp.float32)
  ex = jnp.exp(lg)
  den = acc_ref[0, 0] + acc_ref[1, 0] + ex
  msg = acc_ref[0, 1] + acc_ref[1, 1] + ex * xl
  h = jnp.maximum(msg / (den + 1e-16) + b1_ref[...], 0.0)
  acc = jnp.dot(h, w_ref[...], preferred_element_type=jnp.float32) + b2_ref[...]
  ol_ref[...] = acc[:, :HID]
  or_ref[...] = acc[:, HID:]


def _fuse(acc1, xl1, xr1, att1v, bias1, wcat2, bcat2):
  return pl.pallas_call(
      _fuse_body,
      out_shape=[
          jax.ShapeDtypeStruct((NROWS, HID), jnp.float32),
          jax.ShapeDtypeStruct((NROWS, HID), jnp.float32),
      ],
  )(acc1, xl1, xr1, att1v, _G1, bias1, wcat2, bcat2)


def _final_body(acc_ref, xl_ref, xr_ref, att_ref, g_ref, b_ref, o_ref):
  xl = xl_ref[...]
  lg = jnp.dot(_leaky(xl + xr_ref[...]) * att_ref[...], g_ref[...],
               preferred_element_type=jnp.float32)
  ex = jnp.exp(lg)
  den = acc_ref[0, 0] + acc_ref[1, 0] + ex
  msg = acc_ref[0, 1] + acc_ref[1, 1] + ex * xl
  out = msg / (den + 1e-16) + b_ref[...]
  o_ref[...] = out[:N]


def _final(acc2, xl2, xr2, att2v, bias2):
  return pl.pallas_call(
      _final_body,
      out_shape=jax.ShapeDtypeStruct((N, HID), jnp.float32),
  )(acc2, xl2, xr2, att2v, _G2, bias2)


@jax.jit
def _impl(x, edge_index, Wl1, bl1, Wr1, br1, att1, bias1,
          Wl2, bl2, Wr2, br2, att2, bias2):
  srcp = jnp.pad(edge_index[0], (0, E_PAD - E),
                 constant_values=N).reshape(NW, NBLK, EB)
  dstp = jnp.pad(edge_index[1], (0, E_PAD - E),
                 constant_values=N).reshape(NW, NBLK, EB)

  xpad = jnp.pad(x, ((0, NROWS - N), (0, 0)))
  w1 = jnp.concatenate([Wl1, Wr1], axis=1)
  b1 = jnp.concatenate([bl1, br1]).reshape(1, 2 * HID)
  xl1, xr1 = _proj1(xpad, w1, b1)

  att1v = att1.reshape(1, HID)
  acc1 = _edge_l1(srcp, dstp, xl1, xr1, att1.reshape(HID))

  w2 = jnp.concatenate([Wl2, Wr2], axis=1)
  b2 = jnp.concatenate([bl2, br2]).reshape(1, 2 * HID)
  xl2, xr2 = _fuse(acc1, xl1, xr1, att1v, bias1.reshape(1, HID), w2, b2)

  att2v = att2.reshape(1, HID)
  acc2 = _edge_l2(srcp, dstp, xl2, xr2, att2.reshape(HID))

  return _final(acc2, xl2, xr2, att2v, bias2.reshape(1, HID))


def kernel(x, edge_index, Wl1, bl1, Wr1, br1, att1, bias1,
           Wl2, bl2, Wr2, br2, att2, bias2):
  return _impl(x, edge_index, Wl1, bl1, Wr1, br1, att1, bias1,
               Wl2, bl2, Wr2, br2, att2, bias2)


# speedup vs baseline: 2.0354x; 1.0200x over previous
import functools

import jax
import jax.numpy as jnp
import numpy as np
from jax import lax
from jax.experimental import pallas as pl
from jax.experimental.pallas import tpu as pltpu
from jax.experimental.pallas import tpu_sc as plsc

N = 10000
E = 320000
D_IN = 128
HID = 16

NC = 2
NS = 16
NW = NC * NS
LANES = 16

EB = 128
NBLK = 80
EPW = NBLK * EB
E_PAD = EPW * NW
NROWS = 10240
ROWS_PER_TILE = NROWS // NS
ACC_W = 2 * HID

_GDN = lax.GatherDimensionNumbers(
    offset_dims=(), collapsed_slice_dims=(0,), start_index_map=(0,))


def _shuf(v, perm):
  return lax.gather(v, perm.reshape(LANES, 1), dimension_numbers=_GDN,
                    slice_sizes=(1,),
                    mode=lax.GatherScatterMode.PROMISE_IN_BOUNDS)


def _edge_kernel_body(xors, srcp, dstp, xl, xr, atth, acc_out,
                      attb, sidx, didx, xlb0, xrb0, xlb1, xrb1,
                      exb0, msgb0, exb1, msgb1, zbuf, den_sh, msg_sh,
                      semg0, semg1, sems0, sems1):
  c = lax.axis_index("c")
  s = lax.axis_index("s")
  wid = s * NC + c

  xlbs = (xlb0, xlb1)
  xrbs = (xrb0, xrb1)
  exbs = (exb0, exb1)
  msgbs = (msgb0, msgb1)
  semgs = (semg0, semg1)
  semss = (sems0, sems1)

  zero = jnp.zeros((LANES,), jnp.float32)

  def zb(i, carry):
    zbuf[i, :] = zero
    return carry

  lax.fori_loop(0, ROWS_PER_TILE // 2, zb, 0)
  zbase = s * ROWS_PER_TILE
  half = ROWS_PER_TILE // 2
  pltpu.sync_copy(zbuf, den_sh.at[pl.ds(zbase, half)])
  pltpu.sync_copy(zbuf, den_sh.at[pl.ds(zbase + half, half)])
  pltpu.sync_copy(zbuf, msg_sh.at[pl.ds(zbase, half)])
  pltpu.sync_copy(zbuf, msg_sh.at[pl.ds(zbase + half, half)])
  pltpu.sync_copy(atth, attb)
  pltpu.sync_copy(srcp.at[wid], sidx)
  pltpu.sync_copy(dstp.at[wid], didx)
  plsc.subcore_barrier()

  att = attb[...]
  att5 = att * jnp.float32(0.2)
  perms = [jnp.arange(LANES, dtype=jnp.int32) ^ x for x in xors]

  def gather_start(g, slot):
    pltpu.make_async_copy(xl.at[sidx.at[g]], xlbs[slot], semgs[slot]).start()
    pltpu.make_async_copy(xr.at[didx.at[g]], xrbs[slot], semgs[slot]).start()

  def gather_wait(slot):
    pltpu.make_async_copy(xl.at[sidx.at[0]], xlbs[slot], semgs[slot]).wait()
    pltpu.make_async_copy(xr.at[didx.at[0]], xrbs[slot], semgs[slot]).wait()

  def scatter_start(g, slot):
    pltpu.make_async_copy(exbs[slot], den_sh.at[didx.at[g]],
                          semss[slot]).start(add=True)
    pltpu.make_async_copy(msgbs[slot], msg_sh.at[didx.at[g]],
                          semss[slot]).start(add=True)

  def scatter_wait(slot):
    pltpu.make_async_copy(exbs[slot], den_sh.at[didx.at[0]],
                          semss[slot]).wait()
    pltpu.make_async_copy(msgbs[slot], msg_sh.at[didx.at[0]],
                          semss[slot]).wait()

  def compute(slot):
    xlb, xrb, exb, msgb = xlbs[slot], xrbs[slot], exbs[slot], msgbs[slot]

    @plsc.parallel_loop(0, EB, unroll=4)
    def _edge(i):
      vl = xlb[i, :]
      sv = vl + xrb[i, :]
      p = jnp.where(sv > 0, sv * att, sv * att5)
      for perm in perms:
        p = p + _shuf(p, perm)
      ex = jnp.exp(p)
      exb[i, :] = ex
      msgb[i, :] = ex * vl

  gather_start(0, 0)

  def macro(m, carry):
    g0 = 2 * m
    gather_start(g0 + 1, 1)
    gather_wait(0)

    @pl.when(m > 0)
    def _():
      scatter_wait(0)

    compute(0)
    scatter_start(g0, 0)

    @pl.when(g0 + 2 < NBLK)
    def _():
      gather_start(g0 + 2, 0)

    gather_wait(1)

    @pl.when(m > 0)
    def _():
      scatter_wait(1)

    compute(1)
    scatter_start(g0 + 1, 1)
    return carry

  lax.fori_loop(0, NBLK // 2, macro, 0)
  scatter_wait(0)
  scatter_wait(1)
  plsc.subcore_barrier()

  pltpu.sync_copy(den_sh.at[pl.ds(zbase, ROWS_PER_TILE)],
                  acc_out.at[c, 0, pl.ds(zbase, ROWS_PER_TILE)])
  pltpu.sync_copy(msg_sh.at[pl.ds(zbase, ROWS_PER_TILE)],
                  acc_out.at[c, 1, pl.ds(zbase, ROWS_PER_TILE)])


def _make_edge_kernel(xors):
  mesh = plsc.VectorSubcoreMesh(core_axis_name="c", subcore_axis_name="s",
                                num_cores=NC, num_subcores=NS)
  return pl.kernel(
      functools.partial(_edge_kernel_body, xors),
      out_type=jax.ShapeDtypeStruct((NC, 2, NROWS, HID), jnp.float32),
      mesh=mesh,
      scratch_types=[
          pltpu.VMEM((LANES,), jnp.float32),
          pltpu.VMEM((NBLK, EB), jnp.int32),
          pltpu.VMEM((NBLK, EB), jnp.int32),
          pltpu.VMEM((EB, HID), jnp.float32),
          pltpu.VMEM((EB, HID), jnp.float32),
          pltpu.VMEM((EB, HID), jnp.float32),
          pltpu.VMEM((EB, HID), jnp.float32),
          pltpu.VMEM((EB, HID), jnp.float32),
          pltpu.VMEM((EB, HID), jnp.float32),
          pltpu.VMEM((EB, HID), jnp.float32),
          pltpu.VMEM((EB, HID), jnp.float32),
          pltpu.VMEM((ROWS_PER_TILE // 2, HID), jnp.float32),
          pltpu.VMEM_SHARED((NROWS, HID), jnp.float32),
          pltpu.VMEM_SHARED((NROWS, HID), jnp.float32),
          pltpu.SemaphoreType.DMA,
          pltpu.SemaphoreType.DMA,
          pltpu.SemaphoreType.DMA,
          pltpu.SemaphoreType.DMA,
      ],
      compiler_params=pltpu.CompilerParams(use_tc_tiling_on_sc=False),
      name="gat_edge_pass",
  )


_edge_l1 = _make_edge_kernel((1, 2))
_edge_l2 = _make_edge_kernel((1, 2, 4, 8))

_G1 = np.kron(np.eye(4, dtype=np.float32), np.ones((4, 4), np.float32))
_G2 = np.ones((HID, HID), np.float32)


def _leaky(v):
  return jnp.where(v > 0, v, v * jnp.float32(0.2))


def _proj1_body(x_ref, w_ref, b_ref, ol_ref, or_ref):
  acc = jnp.dot(x_ref[...], w_ref[...],
                preferred_element_type=jnp.float32) + b_ref[...]
  ol_ref[...] = acc[:, :HID]
  or_ref[...] = acc[:, HID:]


def _proj1(xpad, wcat, bcat):
  return pl.pallas_call(
      _proj1_body,
      out_shape=[
          jax.ShapeDtypeStruct((NROWS, HID), jnp.float32),
          jax.ShapeDtypeStruct((NROWS, HID), jnp.float32),
      ],
  )(xpad, wcat, bcat)


def _fuse_body(acc_ref, xl_ref, xr_ref, att_ref, g_ref, b1_ref, w_ref,
               b2_ref, ol_ref, or_ref):
  xl = xl_ref[...]
  lg = jnp.dot(_leaky(xl + xr_ref[...]) * att_ref[...], g_ref[...],
               preferred_element_type=jnp.float32)
  ex = jnp.exp(lg)
  den = acc_ref[0, 0] + acc_ref[1, 0] + ex
  msg = acc_ref[0, 1] + acc_ref[1, 1] + ex * xl
  h = jnp.maximum(msg / (den + 1e-16) + b1_ref[...], 0.0)
  acc = jnp.dot(h, w_ref[...], preferred_element_type=jnp.float32) + b2_ref[...]
  ol_ref[...] = acc[:, :HID]
  or_ref[...] = acc[:, HID:]


def _fuse(acc1, xl1, xr1, att1v, bias1, wcat2, bcat2):
  return pl.pallas_call(
      _fuse_body,
      out_shape=[
          jax.ShapeDtypeStruct((NROWS, HID), jnp.float32),
          jax.ShapeDtypeStruct((NROWS, HID), jnp.float32),
      ],
  )(acc1, xl1, xr1, att1v, _G1, bias1, wcat2, bcat2)


def _final_body(acc_ref, xl_ref, xr_ref, att_ref, g_ref, b_ref, o_ref):
  xl = xl_ref[...]
  lg = jnp.dot(_leaky(xl + xr_ref[...]) * att_ref[...], g_ref[...],
               preferred_element_type=jnp.float32)
  ex = jnp.exp(lg)
  den = acc_ref[0, 0] + acc_ref[1, 0] + ex
  msg = acc_ref[0, 1] + acc_ref[1, 1] + ex * xl
  out = msg / (den + 1e-16) + b_ref[...]
  o_ref[...] = out[:N]


def _final(acc2, xl2, xr2, att2v, bias2):
  return pl.pallas_call(
      _final_body,
      out_shape=jax.ShapeDtypeStruct((N, HID), jnp.float32),
  )(acc2, xl2, xr2, att2v, _G2, bias2)


@jax.jit
def _impl(x, edge_index, Wl1, bl1, Wr1, br1, att1, bias1,
          Wl2, bl2, Wr2, br2, att2, bias2):
  srcp = jnp.pad(edge_index[0], (0, E_PAD - E),
                 constant_values=N).reshape(NW, NBLK, EB)
  dstp = jnp.pad(edge_index[1], (0, E_PAD - E),
                 constant_values=N).reshape(NW, NBLK, EB)

  xpad = jnp.pad(x, ((0, NROWS - N), (0, 0)))
  w1 = jnp.concatenate([Wl1, Wr1], axis=1)
  b1 = jnp.concatenate([bl1, br1]).reshape(1, 2 * HID)
  xl1, xr1 = _proj1(xpad, w1, b1)

  att1v = att1.reshape(1, HID)
  acc1 = _edge_l1(srcp, dstp, xl1, xr1, att1.reshape(HID))

  w2 = jnp.concatenate([Wl2, Wr2], axis=1)
  b2 = jnp.concatenate([bl2, br2]).reshape(1, 2 * HID)
  xl2, xr2 = _fuse(acc1, xl1, xr1, att1v, bias1.reshape(1, HID), w2, b2)

  att2v = att2.reshape(1, HID)
  acc2 = _edge_l2(srcp, dstp, xl2, xr2, att2.reshape(HID))

  return _final(acc2, xl2, xr2, att2v, bias2.reshape(1, HID))


def kernel(x, edge_index, Wl1, bl1, Wr1, br1, att1, bias1,
           Wl2, bl2, Wr2, br2, att2, bias2):
  return _impl(x, edge_index, Wl1, bl1, Wr1, br1, att1, bias1,
               Wl2, bl2, Wr2, br2, att2, bias2)
